# baseline (speedup 1.0000x reference)
import functools

import jax
import jax.numpy as jnp
from jax import lax
from jax.experimental import pallas as pl
from jax.experimental.pallas import tpu as pltpu
from jax.experimental.pallas import tpu_sc as plsc

K = 8192
D = 32
N = 8192
T = 1024
KT = 2048
COMMITMENT_COST = 0.25

_SC_INFO = plsc.get_sparse_core_info()
_NW = _SC_INFO.num_cores * _SC_INFO.num_subcores
_BPW = N // _NW


def _argmin_kernel(x_ref, emb_ref, idx_ref):
    emb = emb_ref[...]
    x = x_ref[...]
    xb = x.astype(jnp.bfloat16)
    e2 = emb * emb
    e_norm = lax.dot_general(jnp.ones((1, D), jnp.float32), e2,
                             (((1,), (1,)), ((), ())),
                             preferred_element_type=jnp.float32,
                             precision=lax.Precision.HIGHEST)
    x_norm = jnp.sum(x * x, axis=1, keepdims=True)
    iota_k = lax.broadcasted_iota(jnp.int32, (1, KT), 1)

    def half_argmin(h0):
        best_d = jnp.full((T, 1), jnp.inf, dtype=jnp.float32)
        best_i = jnp.zeros((T, 1), dtype=jnp.int32)
        for c in range(K // KT // 2):
            k0 = h0 + c * KT
            ec = emb[k0:k0 + KT, :].astype(jnp.bfloat16)
            mm = lax.dot_general(xb, ec, (((1,), (1,)), ((), ())),
                                 preferred_element_type=jnp.float32)
            d = (x_norm + e_norm[:, k0:k0 + KT]) - 2.0 * mm
            lm = jnp.min(d, axis=1, keepdims=True)
            cand = jnp.min(jnp.where(d == lm, iota_k + k0, K),
                           axis=1, keepdims=True)
            upd = lm < best_d
            best_d = jnp.where(upd, lm, best_d)
            best_i = jnp.where(upd, cand, best_i)
        return best_d, best_i

    v_a, i_a = half_argmin(0)
    v_b, i_b = half_argmin(K // 2)
    av_a = v_a.astype(jnp.bfloat16).astype(jnp.float32)
    take_b = (v_b < av_a) | ((v_b == av_a) & (i_b < i_a))
    idx_ref[0, :, :] = jnp.where(take_b, i_b, i_a)


def _encodings_kernel(idx_ref, enc_ref, counts_ref, perp_ref):
    j = pl.program_id(0)
    i = pl.program_id(1)
    idx = idx_ref[0, :, :]
    iota_k = lax.broadcasted_iota(jnp.int32, (1, KT), 1) + i * KT
    sel = (idx == iota_k).astype(jnp.float32)
    enc_ref[...] = sel

    colsum = jnp.sum(sel, axis=0, keepdims=True)
    prev = counts_ref[:, pl.ds(i * KT, KT)]
    counts_ref[:, pl.ds(i * KT, KT)] = jnp.where(j == 0, colsum,
                                                 colsum + prev)

    @pl.when((j == pl.num_programs(0) - 1) & (i == pl.num_programs(1) - 1))
    def _():
        p = counts_ref[...] * (1.0 / N)
        ent = jnp.sum(p * jnp.log(p + 1e-10), axis=(0, 1), keepdims=True)
        perp_ref[...] = jnp.exp(-ent)


def _sc_gather(table_hbm, idx_hbm, x_hbm, qst_hbm, losspart_hbm,
               idx_v, rows_v, x_v, qst_v, acc_v, sem):
    wid = lax.axis_index("s") * _SC_INFO.num_cores + lax.axis_index("c")
    base = wid * _BPW
    for c in range(_BPW // 128):
        pltpu.sync_copy(idx_hbm.at[pl.ds(base + c * 128, 128)], idx_v.at[c])
        pltpu.async_copy(table_hbm.at[idx_v.at[c]],
                         rows_v.at[pl.ds(c * 128, 128), :], sem).wait()
    pltpu.sync_copy(x_hbm.at[pl.ds(base, _BPW), :], x_v)

    def body(r, acc):
        a0, a1 = acc
        q0 = rows_v[r, pl.ds(0, 16)]
        x0 = x_v[r, pl.ds(0, 16)]
        d0 = q0 - x0
        qst_v[r, pl.ds(0, 16)] = x0 + d0
        q1 = rows_v[r, pl.ds(16, 16)]
        x1 = x_v[r, pl.ds(16, 16)]
        d1 = q1 - x1
        qst_v[r, pl.ds(16, 16)] = x1 + d1
        return (a0 + d0 * d0, a1 + d1 * d1)

    z = jnp.zeros((16,), jnp.float32)
    a0, a1 = lax.fori_loop(0, _BPW, body, (z, z))
    acc_v[...] = a0 + a1
    pltpu.sync_copy(qst_v, qst_hbm.at[pl.ds(base, _BPW), :])
    pltpu.sync_copy(acc_v, losspart_hbm.at[wid])


def _loss_kernel(part_ref, loss_ref):
    scale = (1.0 + COMMITMENT_COST) / (N * D)
    loss_ref[...] = jnp.sum(part_ref[...], axis=(0, 1),
                            keepdims=True) * scale


def kernel(inputs, embedding):
    input_shape = inputs.shape
    flat = inputs.reshape(N, D)

    idx3 = pl.pallas_call(
        _argmin_kernel,
        grid=(N // T,),
        in_specs=[
            pl.BlockSpec((T, D), lambda j: (j, 0)),
            pl.BlockSpec((K, D), lambda j: (0, 0)),
        ],
        out_specs=pl.BlockSpec((1, T, 1), lambda j: (j, 0, 0)),
        out_shape=jax.ShapeDtypeStruct((N // T, T, 1), jnp.int32),
    )(flat, embedding)

    encodings, counts, perp = pl.pallas_call(
        _encodings_kernel,
        grid=(N // T, K // KT),
        in_specs=[pl.BlockSpec((1, T, 1), lambda j, i: (j, 0, 0))],
        out_specs=[
            pl.BlockSpec((T, KT), lambda j, i: (j, i)),
            pl.BlockSpec((1, K), lambda j, i: (0, 0)),
            pl.BlockSpec((1, 1), lambda j, i: (0, 0)),
        ],
        out_shape=[
            jax.ShapeDtypeStruct((N, K), jnp.float32),
            jax.ShapeDtypeStruct((1, K), jnp.float32),
            jax.ShapeDtypeStruct((1, 1), jnp.float32),
        ],
    )(idx3)

    original_indices = idx3.reshape(N)
    table = jnp.pad(embedding.astype(jnp.bfloat16).astype(jnp.float32),
                    ((0, 0), (0, 128 - D)))

    sc = functools.partial(
        pl.kernel,
        mesh=plsc.VectorSubcoreMesh(core_axis_name="c", subcore_axis_name="s"),
        out_type=[
            jax.ShapeDtypeStruct((N, D), jnp.float32),
            jax.ShapeDtypeStruct((_NW, 16), jnp.float32),
        ],
        scratch_types=[
            pltpu.VMEM((_BPW // 128, 128), jnp.int32),
            pltpu.VMEM((_BPW, 128), jnp.float32),
            pltpu.VMEM((_BPW, D), jnp.float32),
            pltpu.VMEM((_BPW, D), jnp.float32),
            pltpu.VMEM((16,), jnp.float32),
            pltpu.SemaphoreType.DMA,
        ],
    )(_sc_gather)
    quantized_st, loss_parts = sc(table, original_indices, flat)

    loss = pl.pallas_call(
        _loss_kernel,
        out_shape=jax.ShapeDtypeStruct((1, 1), jnp.float32),
    )(loss_parts)

    vq_loss = loss[0, 0]
    perplexity = perp[0, 0]
    return (vq_loss, quantized_st.reshape(input_shape), perplexity,
            encodings, original_indices)


# speedup vs baseline: 1.1176x; 1.1176x over previous
import functools

import jax
import jax.numpy as jnp
from jax import lax
from jax.experimental import pallas as pl
from jax.experimental.pallas import tpu as pltpu
from jax.experimental.pallas import tpu_sc as plsc

K = 8192
D = 32
N = 8192
T = 1024
KT = 2048
COMMITMENT_COST = 0.25

_SC_INFO = plsc.get_sparse_core_info()
_NW = _SC_INFO.num_cores * _SC_INFO.num_subcores
_BPW = N // _NW


def _argmin_kernel(x_ref, emb_ref, xn_ref, en_ref, idx_ref):
    emb = emb_ref[...]
    x = x_ref[...]
    xb = x.astype(jnp.bfloat16)
    e_norm = en_ref[...]
    x_norm = xn_ref[...]
    iota_k = lax.broadcasted_iota(jnp.int32, (1, KT), 1)

    def half_argmin(h0):
        best_d = jnp.full((T, 1), jnp.inf, dtype=jnp.float32)
        best_i = jnp.zeros((T, 1), dtype=jnp.int32)
        for c in range(K // KT // 2):
            k0 = h0 + c * KT
            ec = emb[k0:k0 + KT, :].astype(jnp.bfloat16)
            mm = lax.dot_general(xb, ec, (((1,), (1,)), ((), ())),
                                 preferred_element_type=jnp.float32)
            d = (x_norm + e_norm[:, k0:k0 + KT]) - 2.0 * mm
            lm = jnp.min(d, axis=1, keepdims=True)
            cand = jnp.min(jnp.where(d == lm, iota_k + k0, K),
                           axis=1, keepdims=True)
            upd = lm < best_d
            best_d = jnp.where(upd, lm, best_d)
            best_i = jnp.where(upd, cand, best_i)
        return best_d, best_i

    v_a, i_a = half_argmin(0)
    v_b, i_b = half_argmin(K // 2)
    av_a = v_a.astype(jnp.bfloat16).astype(jnp.float32)
    take_b = (v_b < av_a) | ((v_b == av_a) & (i_b < i_a))
    idx_ref[0, :, :] = jnp.where(take_b, i_b, i_a)


def _encodings_kernel(idx_ref, enc_ref, counts_ref, perp_ref):
    j = pl.program_id(0)
    i = pl.program_id(1)
    idx = idx_ref[0, :, :]
    iota_k = lax.broadcasted_iota(jnp.int32, (1, KT), 1) + i * KT
    sel = (idx == iota_k).astype(jnp.float32)
    enc_ref[...] = sel

    colsum = jnp.sum(sel, axis=0, keepdims=True)
    prev = counts_ref[:, pl.ds(i * KT, KT)]
    counts_ref[:, pl.ds(i * KT, KT)] = jnp.where(j == 0, colsum,
                                                 colsum + prev)

    @pl.when((j == pl.num_programs(0) - 1) & (i == pl.num_programs(1) - 1))
    def _():
        p = counts_ref[...] * (1.0 / N)
        ent = jnp.sum(p * jnp.log(p + 1e-10), axis=(0, 1), keepdims=True)
        perp_ref[...] = jnp.exp(-ent)


def _sc_gather(table_hbm, idx_hbm, x_hbm, qst_hbm, losspart_hbm,
               idx_v, rows_v, x_v, qst_v, acc_v, sem):
    wid = lax.axis_index("s") * _SC_INFO.num_cores + lax.axis_index("c")
    base = wid * _BPW
    for c in range(_BPW // 128):
        pltpu.sync_copy(idx_hbm.at[pl.ds(base + c * 128, 128)], idx_v.at[c])
        pltpu.async_copy(table_hbm.at[idx_v.at[c]],
                         rows_v.at[pl.ds(c * 128, 128), :], sem).wait()
    pltpu.sync_copy(x_hbm.at[pl.ds(base, _BPW), :], x_v)

    def body(r, acc):
        a0, a1 = acc
        q0 = rows_v[r, pl.ds(0, 16)]
        x0 = x_v[r, pl.ds(0, 16)]
        d0 = q0 - x0
        qst_v[r, pl.ds(0, 16)] = x0 + d0
        q1 = rows_v[r, pl.ds(16, 16)]
        x1 = x_v[r, pl.ds(16, 16)]
        d1 = q1 - x1
        qst_v[r, pl.ds(16, 16)] = x1 + d1
        return (a0 + d0 * d0, a1 + d1 * d1)

    z = jnp.zeros((16,), jnp.float32)
    a0, a1 = lax.fori_loop(0, _BPW, body, (z, z))
    acc_v[...] = a0 + a1
    pltpu.sync_copy(qst_v, qst_hbm.at[pl.ds(base, _BPW), :])
    pltpu.sync_copy(acc_v, losspart_hbm.at[wid])


def _loss_kernel(part_ref, loss_ref):
    scale = (1.0 + COMMITMENT_COST) / (N * D)
    loss_ref[...] = jnp.sum(part_ref[...], axis=(0, 1),
                            keepdims=True) * scale


def kernel(inputs, embedding):
    input_shape = inputs.shape
    flat = inputs.reshape(N, D)

    x_norm = jnp.sum(flat ** 2, axis=1, keepdims=True)
    e_norm = jnp.sum(embedding ** 2, axis=1)[None, :]

    idx3 = pl.pallas_call(
        _argmin_kernel,
        grid=(N // T,),
        in_specs=[
            pl.BlockSpec((T, D), lambda j: (j, 0)),
            pl.BlockSpec((K, D), lambda j: (0, 0)),
            pl.BlockSpec((T, 1), lambda j: (j, 0)),
            pl.BlockSpec((1, K), lambda j: (0, 0)),
        ],
        out_specs=pl.BlockSpec((1, T, 1), lambda j: (j, 0, 0)),
        out_shape=jax.ShapeDtypeStruct((N // T, T, 1), jnp.int32),
    )(flat, embedding, x_norm, e_norm)

    encodings, counts, perp = pl.pallas_call(
        _encodings_kernel,
        grid=(N // T, K // KT),
        in_specs=[pl.BlockSpec((1, T, 1), lambda j, i: (j, 0, 0))],
        out_specs=[
            pl.BlockSpec((T, KT), lambda j, i: (j, i)),
            pl.BlockSpec((1, K), lambda j, i: (0, 0)),
            pl.BlockSpec((1, 1), lambda j, i: (0, 0)),
        ],
        out_shape=[
            jax.ShapeDtypeStruct((N, K), jnp.float32),
            jax.ShapeDtypeStruct((1, K), jnp.float32),
            jax.ShapeDtypeStruct((1, 1), jnp.float32),
        ],
    )(idx3)

    original_indices = idx3.reshape(N)
    table = jnp.pad(embedding.astype(jnp.bfloat16).astype(jnp.float32),
                    ((0, 0), (0, 128 - D)))

    sc = functools.partial(
        pl.kernel,
        mesh=plsc.VectorSubcoreMesh(core_axis_name="c", subcore_axis_name="s"),
        out_type=[
            jax.ShapeDtypeStruct((N, D), jnp.float32),
            jax.ShapeDtypeStruct((_NW, 16), jnp.float32),
        ],
        scratch_types=[
            pltpu.VMEM((_BPW // 128, 128), jnp.int32),
            pltpu.VMEM((_BPW, 128), jnp.float32),
            pltpu.VMEM((_BPW, D), jnp.float32),
            pltpu.VMEM((_BPW, D), jnp.float32),
            pltpu.VMEM((16,), jnp.float32),
            pltpu.SemaphoreType.DMA,
        ],
    )(_sc_gather)
    quantized_st, loss_parts = sc(table, original_indices, flat)

    loss = pl.pallas_call(
        _loss_kernel,
        out_shape=jax.ShapeDtypeStruct((1, 1), jnp.float32),
    )(loss_parts)

    vq_loss = loss[0, 0]
    perplexity = perp[0, 0]
    return (vq_loss, quantized_st.reshape(input_shape), perplexity,
            encodings, original_indices)
